# Initial kernel scaffold; baseline (speedup 1.0000x reference)
import jax
import jax.numpy as jnp
from jax.experimental import pallas as pl


def kernel(x, edge_index, edge_weight, edge_attr, Wf1, bf1, Wf2, bf2, lin1_W, lin2_W, lin2_b, lin_W, lin_b):
    raise NotImplementedError("write your pallas kernel here")


# speedup vs baseline: 1.5307x; 1.5307x over previous
import functools
import math

import jax
import jax.numpy as jnp
from jax import lax
from jax.experimental import pallas as pl
from jax.experimental.pallas import tpu as pltpu
from jax.experimental.pallas import tpu_sc as plsc

N_NODES = 10000
N_EDGES = 320000
HIDDEN = 128
NUM_RBF = 16
CUTOFF = 5.0

NC = 2
NS = 16
NW = NC * NS
E_PER_W = N_EDGES // NW
CHUNK = 80
N_CHUNKS = E_PER_W // CHUNK
N_PAD = 10240
ROWS_PER_TILE = N_PAD // NS
ZROWS = 128
LANES = 16

EDGE_BLK = 6400


def _filter_body(ea_ref, ew_ref, wf1t_ref, bf1_ref, wf2t_ref, bf2_ref, out_ref):
    t = jnp.tanh(jnp.dot(ea_ref[...], wf1t_ref[...],
                         preferred_element_type=jnp.float32) + bf1_ref[...])
    wf = jnp.dot(t, wf2t_ref[...], preferred_element_type=jnp.float32) + bf2_ref[...]
    c = 0.5 * (jnp.cos(ew_ref[...] * (math.pi / CUTOFF)) + 1.0)
    out_ref[...] = wf * c


def _h_body(x_ref, w_ref, out_ref):
    out_ref[...] = jnp.dot(x_ref[...], w_ref[...],
                           preferred_element_type=jnp.float32)


def _tail_body(p_ref, w2_ref, b2_ref, w3_ref, b3_ref, out_ref):
    agg = p_ref[0] + p_ref[1]
    y = jnp.tanh(jnp.dot(agg, w2_ref[...],
                         preferred_element_type=jnp.float32) + b2_ref[...])
    out_ref[...] = jnp.dot(y, w3_ref[...],
                           preferred_element_type=jnp.float32) + b3_ref[...]


def _sc_body(src_hbm, dst_hbm, h_hbm, wf_hbm, out_hbm,
             acc, sidx, didx, rows, wfb, zbuf, sem):
    c = lax.axis_index("c")
    s = lax.axis_index("s")
    wid = s * NC + c

    zero16 = jnp.zeros((LANES,), jnp.float32)

    def _zrow(i, carry):
        for j in range(HIDDEN // LANES):
            zbuf[i, pl.ds(j * LANES, LANES)] = zero16
        return carry

    lax.fori_loop(0, ZROWS, _zrow, 0)
    for k in range(ROWS_PER_TILE // ZROWS):
        pltpu.sync_copy(zbuf, acc.at[pl.ds(s * ROWS_PER_TILE + k * ZROWS, ZROWS)])
    plsc.subcore_barrier()

    def _step(it, carry):
        base = wid * E_PER_W + it * CHUNK
        pltpu.sync_copy(src_hbm.at[pl.ds(base, CHUNK)], sidx)
        pltpu.sync_copy(dst_hbm.at[pl.ds(base, CHUNK)], didx)
        pltpu.async_copy(h_hbm.at[sidx], rows, sem).wait()
        pltpu.sync_copy(wf_hbm.at[pl.ds(base, CHUNK)], wfb)

        def _mul(e, inner):
            for j in range(HIDDEN // LANES):
                sl = pl.ds(j * LANES, LANES)
                rows[e, sl] = rows[e, sl] * wfb[e, sl]
            return inner

        lax.fori_loop(0, CHUNK, _mul, 0)
        pltpu.sync_copy(rows, acc.at[didx], add=True)
        return carry

    lax.fori_loop(0, N_CHUNKS, _step, 0)
    plsc.subcore_barrier()

    for k in range(ROWS_PER_TILE // ZROWS):
        r0 = s * ROWS_PER_TILE + k * ZROWS
        pltpu.sync_copy(acc.at[pl.ds(r0, ZROWS)], zbuf)
        pltpu.sync_copy(zbuf, out_hbm.at[c, pl.ds(r0, ZROWS)])


def kernel(x, edge_index, edge_weight, edge_attr, Wf1, bf1, Wf2, bf2,
           lin1_W, lin2_W, lin2_b, lin_W, lin_b):
    src = edge_index[0].astype(jnp.int32)
    dst = edge_index[1].astype(jnp.int32)
    ew = edge_weight.reshape(N_EDGES, 1)

    wfilt = pl.pallas_call(
        _filter_body,
        grid=(N_EDGES // EDGE_BLK,),
        in_specs=[
            pl.BlockSpec((EDGE_BLK, NUM_RBF), lambda i: (i, 0)),
            pl.BlockSpec((EDGE_BLK, 1), lambda i: (i, 0)),
            pl.BlockSpec((NUM_RBF, HIDDEN), lambda i: (0, 0)),
            pl.BlockSpec((1, HIDDEN), lambda i: (0, 0)),
            pl.BlockSpec((HIDDEN, HIDDEN), lambda i: (0, 0)),
            pl.BlockSpec((1, HIDDEN), lambda i: (0, 0)),
        ],
        out_specs=pl.BlockSpec((EDGE_BLK, HIDDEN), lambda i: (i, 0)),
        out_shape=jax.ShapeDtypeStruct((N_EDGES, HIDDEN), jnp.float32),
    )(edge_attr, ew, Wf1.T, bf1.reshape(1, HIDDEN), Wf2.T, bf2.reshape(1, HIDDEN))

    h = pl.pallas_call(
        _h_body,
        out_shape=jax.ShapeDtypeStruct((N_NODES, HIDDEN), jnp.float32),
    )(x, lin1_W.T)

    mesh = plsc.VectorSubcoreMesh(core_axis_name="c", subcore_axis_name="s")
    partials = pl.kernel(
        _sc_body,
        out_type=jax.ShapeDtypeStruct((NC, N_PAD, HIDDEN), jnp.float32),
        mesh=mesh,
        scratch_types=[
            pltpu.VMEM_SHARED((N_PAD, HIDDEN), jnp.float32),
            pltpu.VMEM((CHUNK,), jnp.int32),
            pltpu.VMEM((CHUNK,), jnp.int32),
            pltpu.VMEM((CHUNK, HIDDEN), jnp.float32),
            pltpu.VMEM((CHUNK, HIDDEN), jnp.float32),
            pltpu.VMEM((ZROWS, HIDDEN), jnp.float32),
            pltpu.SemaphoreType.DMA,
        ],
    )(src, dst, h, wfilt)

    out = pl.pallas_call(
        _tail_body,
        out_shape=jax.ShapeDtypeStruct((N_PAD, HIDDEN), jnp.float32),
    )(partials, lin2_W.T, lin2_b.reshape(1, HIDDEN), lin_W.T,
      lin_b.reshape(1, HIDDEN))
    return out[:N_NODES]


# speedup vs baseline: 1.7582x; 1.1486x over previous
import functools
import math

import jax
import jax.numpy as jnp
from jax import lax
from jax.experimental import pallas as pl
from jax.experimental.pallas import tpu as pltpu
from jax.experimental.pallas import tpu_sc as plsc

N_NODES = 10000
N_EDGES = 320000
HIDDEN = 128
NUM_RBF = 16
CUTOFF = 5.0

NC = 2
NS = 16
NW = NC * NS
E_PER_W = N_EDGES // NW
CHUNK = 40
N_CHUNKS = E_PER_W // CHUNK
NBUF = 3
N_PAD = 10240
ROWS_PER_TILE = N_PAD // NS
ZROWS = 128
LANES = 16

EDGE_BLK = 6400


def _filter_body(ea_ref, ew_ref, wf1t_ref, bf1_ref, wf2t_ref, bf2_ref, out_ref):
    t = jnp.tanh(jnp.dot(ea_ref[...], wf1t_ref[...],
                         preferred_element_type=jnp.float32) + bf1_ref[...])
    wf = jnp.dot(t, wf2t_ref[...], preferred_element_type=jnp.float32) + bf2_ref[...]
    c = 0.5 * (jnp.cos(ew_ref[...] * (math.pi / CUTOFF)) + 1.0)
    out_ref[...] = wf * c


def _h_body(x_ref, w_ref, out_ref):
    out_ref[...] = jnp.dot(x_ref[...], w_ref[...],
                           preferred_element_type=jnp.float32)


def _tail_body(p_ref, w2_ref, b2_ref, w3_ref, b3_ref, out_ref):
    agg = p_ref[0] + p_ref[1]
    y = jnp.tanh(jnp.dot(agg, w2_ref[...],
                         preferred_element_type=jnp.float32) + b2_ref[...])
    out_ref[...] = jnp.dot(y, w3_ref[...],
                           preferred_element_type=jnp.float32) + b3_ref[...]


def _sc_body(src_hbm, dst_hbm, h_hbm, wf_hbm, out_hbm,
             acc, sidx0, sidx1, sidx2, didx0, didx1, didx2,
             rows0, rows1, rows2, wfb0, wfb1, wfb2, zbuf,
             gsem0, gsem1, gsem2, wsem0, wsem1, wsem2,
             ssem0, ssem1, ssem2, isem0, isem1, isem2):
    c = lax.axis_index("c")
    s = lax.axis_index("s")
    wid = s * NC + c
    sidx = (sidx0, sidx1, sidx2)
    didx = (didx0, didx1, didx2)
    rows = (rows0, rows1, rows2)
    wfb = (wfb0, wfb1, wfb2)
    gsem = (gsem0, gsem1, gsem2)
    wsem = (wsem0, wsem1, wsem2)
    ssem = (ssem0, ssem1, ssem2)
    isem = (isem0, isem1, isem2)

    zero16 = jnp.zeros((LANES,), jnp.float32)

    def _zrow(i, carry):
        for j in range(HIDDEN // LANES):
            zbuf[i, pl.ds(j * LANES, LANES)] = zero16
        return carry

    lax.fori_loop(0, ZROWS, _zrow, 0)
    for k in range(ROWS_PER_TILE // ZROWS):
        pltpu.sync_copy(zbuf, acc.at[pl.ds(s * ROWS_PER_TILE + k * ZROWS, ZROWS)])
    plsc.subcore_barrier()

    def _start_idx(it, b):
        pltpu.async_copy(src_hbm.at[wid, it], sidx[b], isem[b])
        pltpu.async_copy(dst_hbm.at[wid, it], didx[b], isem[b])

    def _wait_idx(it, b):
        pltpu.make_async_copy(src_hbm.at[wid, it], sidx[b], isem[b]).wait()
        pltpu.make_async_copy(dst_hbm.at[wid, it], didx[b], isem[b]).wait()

    def _start_loads(it, b):
        pltpu.async_copy(h_hbm.at[sidx[b]], rows[b], gsem[b])
        base = pl.multiple_of(wid * E_PER_W + it * CHUNK, CHUNK)
        pltpu.async_copy(wf_hbm.at[pl.ds(base, CHUNK)], wfb[b], wsem[b])

    def _wait_loads(it, b):
        pltpu.make_async_copy(h_hbm.at[sidx[b]], rows[b], gsem[b]).wait()
        base = pl.multiple_of(wid * E_PER_W + it * CHUNK, CHUNK)
        pltpu.make_async_copy(wf_hbm.at[pl.ds(base, CHUNK)], wfb[b], wsem[b]).wait()

    def _mul(b):
        def body(e, carry):
            for j in range(HIDDEN // LANES):
                sl = pl.ds(j * LANES, LANES)
                rows[b][e, sl] = rows[b][e, sl] * wfb[b][e, sl]
            return carry
        lax.fori_loop(0, CHUNK, body, 0)

    def _start_scatter(it, b):
        pltpu.async_copy(rows[b], acc.at[didx[b]], ssem[b], add=True)

    def _wait_scatter(b):
        pltpu.make_async_copy(rows[b], acc.at[didx[b]], ssem[b]).wait()

    def _step(it, b, drain, nxt):
        bn = (b + 1) % NBUF
        if drain:
            _wait_scatter(bn)
        if nxt:
            _start_idx(it + 1, bn)
        _wait_loads(it, b)
        _mul(b)
        if nxt:
            _wait_idx(it + 1, bn)
            _start_loads(it + 1, bn)
        _start_scatter(it, b)

    _start_idx(0, 0)
    _wait_idx(0, 0)
    _start_loads(0, 0)
    _step(0, 0, drain=False, nxt=True)
    _step(1, 1, drain=False, nxt=True)

    def _outer(g, carry):
        it0 = NBUF * g + 2
        for d in range(NBUF):
            _step(it0 + d, (2 + d) % NBUF, drain=True, nxt=True)
        return carry

    n_steady = (N_CHUNKS - 4) // NBUF
    lax.fori_loop(0, n_steady, _outer, 0)
    _step(N_CHUNKS - 2, (N_CHUNKS - 2) % NBUF, drain=True, nxt=True)
    _step(N_CHUNKS - 1, (N_CHUNKS - 1) % NBUF, drain=True, nxt=False)

    for it in range(N_CHUNKS - NBUF + 1, N_CHUNKS):
        _wait_scatter(it % NBUF)
    plsc.subcore_barrier()

    for k in range(ROWS_PER_TILE // ZROWS):
        r0 = s * ROWS_PER_TILE + k * ZROWS
        pltpu.sync_copy(acc.at[pl.ds(r0, ZROWS)], zbuf)
        pltpu.sync_copy(zbuf, out_hbm.at[c, pl.ds(r0, ZROWS)])


def kernel(x, edge_index, edge_weight, edge_attr, Wf1, bf1, Wf2, bf2,
           lin1_W, lin2_W, lin2_b, lin_W, lin_b):
    src = edge_index[0].astype(jnp.int32).reshape(NW, N_CHUNKS, CHUNK)
    dst = edge_index[1].astype(jnp.int32).reshape(NW, N_CHUNKS, CHUNK)
    ew = edge_weight.reshape(N_EDGES, 1)

    wfilt = pl.pallas_call(
        _filter_body,
        grid=(N_EDGES // EDGE_BLK,),
        in_specs=[
            pl.BlockSpec((EDGE_BLK, NUM_RBF), lambda i: (i, 0)),
            pl.BlockSpec((EDGE_BLK, 1), lambda i: (i, 0)),
            pl.BlockSpec((NUM_RBF, HIDDEN), lambda i: (0, 0)),
            pl.BlockSpec((1, HIDDEN), lambda i: (0, 0)),
            pl.BlockSpec((HIDDEN, HIDDEN), lambda i: (0, 0)),
            pl.BlockSpec((1, HIDDEN), lambda i: (0, 0)),
        ],
        out_specs=pl.BlockSpec((EDGE_BLK, HIDDEN), lambda i: (i, 0)),
        out_shape=jax.ShapeDtypeStruct((N_EDGES, HIDDEN), jnp.float32),
    )(edge_attr, ew, Wf1.T, bf1.reshape(1, HIDDEN), Wf2.T, bf2.reshape(1, HIDDEN))

    h = pl.pallas_call(
        _h_body,
        out_shape=jax.ShapeDtypeStruct((N_NODES, HIDDEN), jnp.float32),
    )(x, lin1_W.T)

    mesh = plsc.VectorSubcoreMesh(core_axis_name="c", subcore_axis_name="s")
    partials = pl.kernel(
        _sc_body,
        out_type=jax.ShapeDtypeStruct((NC, N_PAD, HIDDEN), jnp.float32),
        mesh=mesh,
        scratch_types=[
            pltpu.VMEM_SHARED((N_PAD, HIDDEN), jnp.float32),
            pltpu.VMEM((CHUNK,), jnp.int32),
            pltpu.VMEM((CHUNK,), jnp.int32),
            pltpu.VMEM((CHUNK,), jnp.int32),
            pltpu.VMEM((CHUNK,), jnp.int32),
            pltpu.VMEM((CHUNK,), jnp.int32),
            pltpu.VMEM((CHUNK,), jnp.int32),
            pltpu.VMEM((CHUNK, HIDDEN), jnp.float32),
            pltpu.VMEM((CHUNK, HIDDEN), jnp.float32),
            pltpu.VMEM((CHUNK, HIDDEN), jnp.float32),
            pltpu.VMEM((CHUNK, HIDDEN), jnp.float32),
            pltpu.VMEM((CHUNK, HIDDEN), jnp.float32),
            pltpu.VMEM((CHUNK, HIDDEN), jnp.float32),
            pltpu.VMEM((ZROWS, HIDDEN), jnp.float32),
        ] + [pltpu.SemaphoreType.DMA] * 12,
    )(src, dst, h, wfilt)

    out = pl.pallas_call(
        _tail_body,
        out_shape=jax.ShapeDtypeStruct((N_PAD, HIDDEN), jnp.float32),
    )(partials, lin2_W.T, lin2_b.reshape(1, HIDDEN), lin_W.T,
      lin_b.reshape(1, HIDDEN))
    return out[:N_NODES]


# speedup vs baseline: 2.6744x; 1.5211x over previous
import functools
import math

import jax
import jax.numpy as jnp
from jax import lax
from jax.experimental import pallas as pl
from jax.experimental.pallas import tpu as pltpu
from jax.experimental.pallas import tpu_sc as plsc

N_NODES = 10000
N_EDGES = 320000
HIDDEN = 128
NUM_RBF = 16
CUTOFF = 5.0

NC = 2
NS = 16
NW = NC * NS
E_PER_W = N_EDGES // NW
CHUNK = 40
N_CHUNKS = E_PER_W // CHUNK
NBUF = 3
N_PAD = 10240
ROWS_PER_TILE = N_PAD // NS
ZROWS = 128
LANES = 16

EDGE_BLK = 12800


def _filter_body(ea_ref, ew_ref, wf1t_ref, bf1_ref, wf2t_ref, bf2_ref, out_ref):
    t = jnp.tanh(jnp.dot(ea_ref[...], wf1t_ref[...],
                         preferred_element_type=jnp.float32) + bf1_ref[...])
    wf = jnp.dot(t, wf2t_ref[...], preferred_element_type=jnp.float32) + bf2_ref[...]
    z2 = jnp.square(ew_ref[...] * (math.pi / (2.0 * CUTOFF)))
    p = 1.0 + z2 * (-1.0 / 2.0 + z2 * (1.0 / 24.0 + z2 * (
        -1.0 / 720.0 + z2 * (1.0 / 40320.0 + z2 * (-1.0 / 3628800.0)))))
    out_ref[...] = wf * (p * p)


def _h_body(x_ref, w_ref, out_ref):
    out_ref[...] = jnp.dot(x_ref[...], w_ref[...],
                           preferred_element_type=jnp.float32)


def _tail_body(p_ref, w2_ref, b2_ref, w3_ref, b3_ref, out_ref):
    agg = p_ref[0] + p_ref[1]
    y = jnp.tanh(jnp.dot(agg, w2_ref[...],
                         preferred_element_type=jnp.float32) + b2_ref[...])
    out_ref[...] = jnp.dot(y, w3_ref[...],
                           preferred_element_type=jnp.float32) + b3_ref[...]


def _sc_body(src_hbm, dst_hbm, h_hbm, wf_hbm, out_hbm,
             acc, sidx0, sidx1, sidx2, didx0, didx1, didx2,
             rows0, rows1, rows2, wfb0, wfb1, wfb2, zbuf,
             gsem0, gsem1, gsem2, wsem0, wsem1, wsem2,
             ssem0, ssem1, ssem2, isem0, isem1, isem2):
    c = lax.axis_index("c")
    s = lax.axis_index("s")
    wid = s * NC + c
    sidx = (sidx0, sidx1, sidx2)
    didx = (didx0, didx1, didx2)
    rows = (rows0, rows1, rows2)
    wfb = (wfb0, wfb1, wfb2)
    gsem = (gsem0, gsem1, gsem2)
    wsem = (wsem0, wsem1, wsem2)
    ssem = (ssem0, ssem1, ssem2)
    isem = (isem0, isem1, isem2)

    zero16 = jnp.zeros((LANES,), jnp.float32)

    def _zrow(i, carry):
        for j in range(HIDDEN // LANES):
            zbuf[i, pl.ds(j * LANES, LANES)] = zero16
        return carry

    lax.fori_loop(0, ZROWS, _zrow, 0)
    for k in range(ROWS_PER_TILE // ZROWS):
        pltpu.sync_copy(zbuf, acc.at[pl.ds(s * ROWS_PER_TILE + k * ZROWS, ZROWS)])
    plsc.subcore_barrier()

    def _start_idx(it, b):
        base = pl.multiple_of(wid * E_PER_W + it * CHUNK, CHUNK)
        pltpu.async_copy(src_hbm.at[pl.ds(base, CHUNK)], sidx[b], isem[b])
        pltpu.async_copy(dst_hbm.at[pl.ds(base, CHUNK)], didx[b], isem[b])

    def _wait_idx(it, b):
        base = pl.multiple_of(wid * E_PER_W + it * CHUNK, CHUNK)
        pltpu.make_async_copy(src_hbm.at[pl.ds(base, CHUNK)], sidx[b], isem[b]).wait()
        pltpu.make_async_copy(dst_hbm.at[pl.ds(base, CHUNK)], didx[b], isem[b]).wait()

    def _start_loads(it, b):
        pltpu.async_copy(h_hbm.at[sidx[b]], rows[b], gsem[b])
        base = pl.multiple_of(wid * E_PER_W + it * CHUNK, CHUNK)
        pltpu.async_copy(wf_hbm.at[pl.ds(base, CHUNK)], wfb[b], wsem[b])

    def _wait_loads(it, b):
        pltpu.make_async_copy(h_hbm.at[sidx[b]], rows[b], gsem[b]).wait()
        base = pl.multiple_of(wid * E_PER_W + it * CHUNK, CHUNK)
        pltpu.make_async_copy(wf_hbm.at[pl.ds(base, CHUNK)], wfb[b], wsem[b]).wait()

    def _mul(b):
        def body(e, carry):
            for j in range(HIDDEN // LANES):
                sl = pl.ds(j * LANES, LANES)
                rows[b][e, sl] = rows[b][e, sl] * wfb[b][e, sl]
            return carry
        lax.fori_loop(0, CHUNK, body, 0)

    def _start_scatter(it, b):
        pltpu.async_copy(rows[b], acc.at[didx[b]], ssem[b], add=True)

    def _wait_scatter(b):
        pltpu.make_async_copy(rows[b], acc.at[didx[b]], ssem[b]).wait()

    def _step(it, b, drain, nxt):
        bn = (b + 1) % NBUF
        if drain:
            _wait_scatter(bn)
        if nxt:
            _start_idx(it + 1, bn)
        _wait_loads(it, b)
        _mul(b)
        if nxt:
            _wait_idx(it + 1, bn)
            _start_loads(it + 1, bn)
        _start_scatter(it, b)

    _start_idx(0, 0)
    _wait_idx(0, 0)
    _start_loads(0, 0)
    _step(0, 0, drain=False, nxt=True)
    _step(1, 1, drain=False, nxt=True)

    def _outer(g, carry):
        it0 = NBUF * g + 2
        for d in range(NBUF):
            _step(it0 + d, (2 + d) % NBUF, drain=True, nxt=True)
        return carry

    n_steady = (N_CHUNKS - 4) // NBUF
    lax.fori_loop(0, n_steady, _outer, 0)
    _step(N_CHUNKS - 2, (N_CHUNKS - 2) % NBUF, drain=True, nxt=True)
    _step(N_CHUNKS - 1, (N_CHUNKS - 1) % NBUF, drain=True, nxt=False)

    for it in range(N_CHUNKS - NBUF + 1, N_CHUNKS):
        _wait_scatter(it % NBUF)
    plsc.subcore_barrier()

    for k in range(ROWS_PER_TILE // ZROWS):
        r0 = s * ROWS_PER_TILE + k * ZROWS
        pltpu.sync_copy(acc.at[pl.ds(r0, ZROWS)], zbuf)
        pltpu.sync_copy(zbuf, out_hbm.at[c, pl.ds(r0, ZROWS)])


def kernel(x, edge_index, edge_weight, edge_attr, Wf1, bf1, Wf2, bf2,
           lin1_W, lin2_W, lin2_b, lin_W, lin_b):
    src = edge_index[0].astype(jnp.int32)
    dst = edge_index[1].astype(jnp.int32)
    ew = edge_weight.reshape(N_EDGES, 1)

    wfilt = pl.pallas_call(
        _filter_body,
        grid=(N_EDGES // EDGE_BLK,),
        in_specs=[
            pl.BlockSpec((EDGE_BLK, NUM_RBF), lambda i: (i, 0)),
            pl.BlockSpec((EDGE_BLK, 1), lambda i: (i, 0)),
            pl.BlockSpec((NUM_RBF, HIDDEN), lambda i: (0, 0)),
            pl.BlockSpec((1, HIDDEN), lambda i: (0, 0)),
            pl.BlockSpec((HIDDEN, HIDDEN), lambda i: (0, 0)),
            pl.BlockSpec((1, HIDDEN), lambda i: (0, 0)),
        ],
        out_specs=pl.BlockSpec((EDGE_BLK, HIDDEN), lambda i: (i, 0)),
        out_shape=jax.ShapeDtypeStruct((N_EDGES, HIDDEN), jnp.float32),
    )(edge_attr, ew, Wf1.T, bf1.reshape(1, HIDDEN), Wf2.T, bf2.reshape(1, HIDDEN))

    h = pl.pallas_call(
        _h_body,
        out_shape=jax.ShapeDtypeStruct((N_NODES, HIDDEN), jnp.float32),
    )(x, lin1_W.T)

    mesh = plsc.VectorSubcoreMesh(core_axis_name="c", subcore_axis_name="s")
    partials = pl.kernel(
        _sc_body,
        out_type=jax.ShapeDtypeStruct((NC, N_PAD, HIDDEN), jnp.float32),
        mesh=mesh,
        scratch_types=[
            pltpu.VMEM_SHARED((N_PAD, HIDDEN), jnp.float32),
            pltpu.VMEM((CHUNK,), jnp.int32),
            pltpu.VMEM((CHUNK,), jnp.int32),
            pltpu.VMEM((CHUNK,), jnp.int32),
            pltpu.VMEM((CHUNK,), jnp.int32),
            pltpu.VMEM((CHUNK,), jnp.int32),
            pltpu.VMEM((CHUNK,), jnp.int32),
            pltpu.VMEM((CHUNK, HIDDEN), jnp.float32),
            pltpu.VMEM((CHUNK, HIDDEN), jnp.float32),
            pltpu.VMEM((CHUNK, HIDDEN), jnp.float32),
            pltpu.VMEM((CHUNK, HIDDEN), jnp.float32),
            pltpu.VMEM((CHUNK, HIDDEN), jnp.float32),
            pltpu.VMEM((CHUNK, HIDDEN), jnp.float32),
            pltpu.VMEM((ZROWS, HIDDEN), jnp.float32),
        ] + [pltpu.SemaphoreType.DMA] * 12,
    )(src, dst, h, wfilt)

    out = pl.pallas_call(
        _tail_body,
        out_shape=jax.ShapeDtypeStruct((N_PAD, HIDDEN), jnp.float32),
    )(partials, lin2_W.T, lin2_b.reshape(1, HIDDEN), lin_W.T,
      lin_b.reshape(1, HIDDEN))
    return out[:N_NODES]


# speedup vs baseline: 4.1259x; 1.5427x over previous
import functools
import math

import jax
import jax.numpy as jnp
from jax import lax
from jax.experimental import pallas as pl
from jax.experimental.pallas import tpu as pltpu
from jax.experimental.pallas import tpu_sc as plsc

N_NODES = 10000
N_EDGES = 320000
HIDDEN = 128
NUM_RBF = 16
CUTOFF = 5.0

NC = 2
NS = 16
NW = NC * NS
E_PER_W = N_EDGES // NW
CHUNK = 40
N_CHUNKS = E_PER_W // CHUNK
NBUF = 3
N_PAD = 10240
ROWS_PER_TILE = N_PAD // NS
ZROWS = 128
LANES = 16

EDGE_BLK = 12800


def _filter_body(eaT_ref, ew_ref, wf1_ref, bf1_ref, wf2t_ref, bf2_ref, out_ref):
    tT = jnp.tanh(jnp.dot(wf1_ref[...], eaT_ref[...],
                          preferred_element_type=jnp.float32) + bf1_ref[...])
    z2 = jnp.square(ew_ref[...] * (math.pi / (2.0 * CUTOFF)))
    p = 1.0 + z2 * (-1.0 / 2.0 + z2 * (1.0 / 24.0 + z2 * (
        -1.0 / 720.0 + z2 * (1.0 / 40320.0 + z2 * (-1.0 / 3628800.0)))))
    c2 = p * p
    tCT = tT * c2
    wf = lax.dot_general(tCT, wf2t_ref[...], (((0,), (0,)), ((), ())),
                         preferred_element_type=jnp.float32)
    wf = wf + lax.dot_general(c2, bf2_ref[...], (((0,), (0,)), ((), ())),
                              preferred_element_type=jnp.float32)
    out_ref[...] = wf


def _h_body(x_ref, w_ref, out_ref):
    out_ref[...] = jnp.dot(x_ref[...], w_ref[...],
                           preferred_element_type=jnp.float32)


def _tail_body(p_ref, w2_ref, b2_ref, w3_ref, b3_ref, out_ref):
    agg = p_ref[0] + p_ref[1]
    y = jnp.tanh(jnp.dot(agg, w2_ref[...],
                         preferred_element_type=jnp.float32) + b2_ref[...])
    out_ref[...] = jnp.dot(y, w3_ref[...],
                           preferred_element_type=jnp.float32) + b3_ref[...]


def _sc_body(src_hbm, dst_hbm, h_hbm, wf_hbm, out_hbm,
             acc, sidx0, sidx1, sidx2, didx0, didx1, didx2,
             rows0, rows1, rows2, wfb0, wfb1, wfb2, zbuf,
             gsem0, gsem1, gsem2, wsem0, wsem1, wsem2,
             ssem0, ssem1, ssem2, isem0, isem1, isem2):
    c = lax.axis_index("c")
    s = lax.axis_index("s")
    wid = s * NC + c
    sidx = (sidx0, sidx1, sidx2)
    didx = (didx0, didx1, didx2)
    rows = (rows0, rows1, rows2)
    wfb = (wfb0, wfb1, wfb2)
    gsem = (gsem0, gsem1, gsem2)
    wsem = (wsem0, wsem1, wsem2)
    ssem = (ssem0, ssem1, ssem2)
    isem = (isem0, isem1, isem2)

    zero16 = jnp.zeros((LANES,), jnp.float32)

    def _zrow(i, carry):
        for j in range(HIDDEN // LANES):
            zbuf[i, pl.ds(j * LANES, LANES)] = zero16
        return carry

    lax.fori_loop(0, ZROWS, _zrow, 0)
    for k in range(ROWS_PER_TILE // ZROWS):
        pltpu.sync_copy(zbuf, acc.at[pl.ds(s * ROWS_PER_TILE + k * ZROWS, ZROWS)])
    plsc.subcore_barrier()

    def _start_idx(it, b):
        base = pl.multiple_of(wid * E_PER_W + it * CHUNK, CHUNK)
        pltpu.async_copy(src_hbm.at[pl.ds(base, CHUNK)], sidx[b], isem[b])
        pltpu.async_copy(dst_hbm.at[pl.ds(base, CHUNK)], didx[b], isem[b])

    def _wait_idx(it, b):
        base = pl.multiple_of(wid * E_PER_W + it * CHUNK, CHUNK)
        pltpu.make_async_copy(src_hbm.at[pl.ds(base, CHUNK)], sidx[b], isem[b]).wait()
        pltpu.make_async_copy(dst_hbm.at[pl.ds(base, CHUNK)], didx[b], isem[b]).wait()

    def _start_loads(it, b):
        pltpu.async_copy(h_hbm.at[sidx[b]], rows[b], gsem[b])
        base = pl.multiple_of(wid * E_PER_W + it * CHUNK, CHUNK)
        pltpu.async_copy(wf_hbm.at[pl.ds(base, CHUNK)], wfb[b], wsem[b])

    def _wait_loads(it, b):
        pltpu.make_async_copy(h_hbm.at[sidx[b]], rows[b], gsem[b]).wait()
        base = pl.multiple_of(wid * E_PER_W + it * CHUNK, CHUNK)
        pltpu.make_async_copy(wf_hbm.at[pl.ds(base, CHUNK)], wfb[b], wsem[b]).wait()

    def _mul(b):
        def body(e, carry):
            for j in range(HIDDEN // LANES):
                sl = pl.ds(j * LANES, LANES)
                rows[b][e, sl] = rows[b][e, sl] * wfb[b][e, sl]
            return carry
        lax.fori_loop(0, CHUNK, body, 0)

    def _start_scatter(it, b):
        pltpu.async_copy(rows[b], acc.at[didx[b]], ssem[b], add=True)

    def _wait_scatter(b):
        pltpu.make_async_copy(rows[b], acc.at[didx[b]], ssem[b]).wait()

    def _step(it, b, drain, nxt):
        bn = (b + 1) % NBUF
        if drain:
            _wait_scatter(bn)
        if nxt:
            _start_idx(it + 1, bn)
        _wait_loads(it, b)
        _mul(b)
        if nxt:
            _wait_idx(it + 1, bn)
            _start_loads(it + 1, bn)
        _start_scatter(it, b)

    _start_idx(0, 0)
    _wait_idx(0, 0)
    _start_loads(0, 0)
    _step(0, 0, drain=False, nxt=True)
    _step(1, 1, drain=False, nxt=True)

    def _outer(g, carry):
        it0 = NBUF * g + 2
        for d in range(NBUF):
            _step(it0 + d, (2 + d) % NBUF, drain=True, nxt=True)
        return carry

    n_steady = (N_CHUNKS - 4) // NBUF
    lax.fori_loop(0, n_steady, _outer, 0)
    _step(N_CHUNKS - 2, (N_CHUNKS - 2) % NBUF, drain=True, nxt=True)
    _step(N_CHUNKS - 1, (N_CHUNKS - 1) % NBUF, drain=True, nxt=False)

    for it in range(N_CHUNKS - NBUF + 1, N_CHUNKS):
        _wait_scatter(it % NBUF)
    plsc.subcore_barrier()

    for k in range(ROWS_PER_TILE // ZROWS):
        r0 = s * ROWS_PER_TILE + k * ZROWS
        pltpu.sync_copy(acc.at[pl.ds(r0, ZROWS)], zbuf)
        pltpu.sync_copy(zbuf, out_hbm.at[c, pl.ds(r0, ZROWS)])


def kernel(x, edge_index, edge_weight, edge_attr, Wf1, bf1, Wf2, bf2,
           lin1_W, lin2_W, lin2_b, lin_W, lin_b):
    src = edge_index[0].astype(jnp.int32)
    dst = edge_index[1].astype(jnp.int32)
    ew = edge_weight.reshape(1, N_EDGES)

    wfilt = pl.pallas_call(
        _filter_body,
        grid=(N_EDGES // EDGE_BLK,),
        in_specs=[
            pl.BlockSpec((NUM_RBF, EDGE_BLK), lambda i: (0, i)),
            pl.BlockSpec((1, EDGE_BLK), lambda i: (0, i)),
            pl.BlockSpec((HIDDEN, NUM_RBF), lambda i: (0, 0)),
            pl.BlockSpec((HIDDEN, 1), lambda i: (0, 0)),
            pl.BlockSpec((HIDDEN, HIDDEN), lambda i: (0, 0)),
            pl.BlockSpec((1, HIDDEN), lambda i: (0, 0)),
        ],
        out_specs=pl.BlockSpec((EDGE_BLK, HIDDEN), lambda i: (i, 0)),
        out_shape=jax.ShapeDtypeStruct((N_EDGES, HIDDEN), jnp.float32),
    )(edge_attr.T, ew, Wf1, bf1.reshape(HIDDEN, 1), Wf2.T, bf2.reshape(1, HIDDEN))

    h = pl.pallas_call(
        _h_body,
        out_shape=jax.ShapeDtypeStruct((N_NODES, HIDDEN), jnp.float32),
    )(x, lin1_W.T)

    mesh = plsc.VectorSubcoreMesh(core_axis_name="c", subcore_axis_name="s")
    partials = pl.kernel(
        _sc_body,
        out_type=jax.ShapeDtypeStruct((NC, N_PAD, HIDDEN), jnp.float32),
        mesh=mesh,
        scratch_types=[
            pltpu.VMEM_SHARED((N_PAD, HIDDEN), jnp.float32),
            pltpu.VMEM((CHUNK,), jnp.int32),
            pltpu.VMEM((CHUNK,), jnp.int32),
            pltpu.VMEM((CHUNK,), jnp.int32),
            pltpu.VMEM((CHUNK,), jnp.int32),
            pltpu.VMEM((CHUNK,), jnp.int32),
            pltpu.VMEM((CHUNK,), jnp.int32),
            pltpu.VMEM((CHUNK, HIDDEN), jnp.float32),
            pltpu.VMEM((CHUNK, HIDDEN), jnp.float32),
            pltpu.VMEM((CHUNK, HIDDEN), jnp.float32),
            pltpu.VMEM((CHUNK, HIDDEN), jnp.float32),
            pltpu.VMEM((CHUNK, HIDDEN), jnp.float32),
            pltpu.VMEM((CHUNK, HIDDEN), jnp.float32),
            pltpu.VMEM((ZROWS, HIDDEN), jnp.float32),
        ] + [pltpu.SemaphoreType.DMA] * 12,
    )(src, dst, h, wfilt)

    out = pl.pallas_call(
        _tail_body,
        out_shape=jax.ShapeDtypeStruct((N_PAD, HIDDEN), jnp.float32),
    )(partials, lin2_W.T, lin2_b.reshape(1, HIDDEN), lin_W.T,
      lin_b.reshape(1, HIDDEN))
    return out[:N_NODES]
